# baseline (device time: 35656 ns/iter reference)
import jax
import jax.numpy as jnp
from jax import lax
from jax.experimental import pallas as pl
from jax.experimental.pallas import tpu as pltpu

N_DEV = 16
B = 2
SQ = 128
D = 512
HQ_LOC = 8
DH = 64
GQA = 4
HKV = HQ_LOC // GQA
R = B * SQ
DC = D // 2


def kernel(x, Wq, Wo, K_ext, V_ext):
    idx = lax.axis_index("i")
    K_loc = jnp.reshape(
        lax.dynamic_slice_in_dim(K_ext, idx * HKV, HKV, axis=2), (B, SQ, HKV * DH))
    V_loc = jnp.reshape(
        lax.dynamic_slice_in_dim(V_ext, idx * HKV, HKV, axis=2), (B, SQ, HKV * DH))

    def body(x_any, wq_any, wo_any, k_any, v_any, out_any,
             x_v, wq_v, wo_v, kk, vv, att_ref,
             pstA, w1A, w2A, w3A, w4A, r0A, r1A, r2A, r3A, g3A, g2A, g1A, g0A,
             pstB, w1B, w2B, w3B, w4B, r0B, r1B, r2B, r3B, g3B, g2B, g1B, g0B,
             in_sems, out_sems,
             rs_sendA, rs_recvA, ag_sendA, ag_recvA,
             rs_sendB, rs_recvB, ag_sendB, ag_recvB):
        my = lax.axis_index("i")

        barrier_sem = pltpu.get_barrier_semaphore()
        for d in (1, 2, 4, 8):
            pl.semaphore_signal(
                barrier_sem, inc=1,
                device_id=(my ^ d,), device_id_type=pl.DeviceIdType.MESH,
            )

        cp_x = pltpu.make_async_copy(x_any, x_v, in_sems.at[0])
        cp_wq = pltpu.make_async_copy(wq_any, wq_v, in_sems.at[1])
        cp_wo = pltpu.make_async_copy(wo_any, wo_v, in_sems.at[2])
        cp_k = pltpu.make_async_copy(k_any, kk, in_sems.at[3])
        cp_v = pltpu.make_async_copy(v_any, vv, in_sems.at[4])
        cp_x.start()
        cp_wq.start()
        cp_wo.start()
        cp_k.start()
        cp_v.start()

        cp_x.wait()
        cp_wq.wait()
        cp_k.wait()
        cp_v.wait()
        for b in range(B):
            qb = jnp.dot(x_v[b], wq_v[...],
                         preferred_element_type=jnp.float32)
            for h in range(HQ_LOC):
                c = h // GQA
                kb = kk[b, :, c * DH:(c + 1) * DH]
                vb = vv[b, :, c * DH:(c + 1) * DH]
                qh = qb[:, h * DH:(h + 1) * DH]
                s = lax.dot_general(
                    qh, kb, (((1,), (1,)), ((), ())),
                    preferred_element_type=jnp.float32,
                ) * 0.125
                m = jnp.max(s, axis=-1, keepdims=True)
                p = jnp.exp(s - m)
                l = jnp.sum(p, axis=-1, keepdims=True)
                o = jnp.dot(p, vb, preferred_element_type=jnp.float32) / l
                att_ref[b, :, h * DH:(h + 1) * DH] = o
            if b == 0:
                cp_wo.wait()
            part = jnp.dot(att_ref[b], wo_v[...],
                           preferred_element_type=jnp.float32)
            pstA[pl.ds(b * SQ, SQ), :] = part[:, :DC]
            pstB[pl.ds(b * SQ, SQ), :] = part[:, DC:]

        pl.semaphore_wait(barrier_sem, 4)

        def bit_is0(d):
            return lax.rem(my // d, 2) == 0

        drains = []

        def rs_start(w_in, rst, S, d, send_sem, recv_sem):
            half = S // 2
            bit0 = bit_is0(d)
            @pl.when(bit0)
            def _():
                pltpu.make_async_remote_copy(
                    src_ref=w_in.at[pl.ds(half, half)], dst_ref=rst,
                    send_sem=send_sem, recv_sem=recv_sem,
                    device_id=(my ^ d,), device_id_type=pl.DeviceIdType.MESH,
                ).start()
            @pl.when(jnp.logical_not(bit0))
            def _():
                pltpu.make_async_remote_copy(
                    src_ref=w_in.at[pl.ds(0, half)], dst_ref=rst,
                    send_sem=send_sem, recv_sem=recv_sem,
                    device_id=(my ^ d,), device_id_type=pl.DeviceIdType.MESH,
                ).start()
            wd = pltpu.make_async_remote_copy(
                src_ref=rst, dst_ref=rst,
                send_sem=send_sem, recv_sem=recv_sem,
                device_id=(my ^ d,), device_id_type=pl.DeviceIdType.MESH,
            )
            return wd, bit0

        def rs_finish(wd, bit0, w_in, rst, w_out, S):
            half = S // 2
            wd.wait_recv()
            lo = w_in[pl.ds(0, half), :]
            hi = w_in[pl.ds(half, half), :]
            w_out[...] = jnp.where(bit0, lo, hi) + rst[...]
            drains.append(wd)

        RS_A = [(pstA, r0A, w1A, R, 1), (w1A, r1A, w2A, R // 2, 2),
                (w2A, r2A, w3A, R // 4, 4), (w3A, r3A, w4A, R // 8, 8)]
        RS_B = [(pstB, r0B, w1B, R, 4), (w1B, r1B, w2B, R // 2, 8),
                (w2B, r2B, w3B, R // 4, 1), (w3B, r3B, w4B, R // 8, 2)]
        for s in range(4):
            w_inA, rstA, w_outA, SA, dA = RS_A[s]
            w_inB, rstB, w_outB, SB, dB = RS_B[s]
            wdA, bitA = rs_start(w_inA, rstA, SA, dA,
                                 rs_sendA.at[s], rs_recvA.at[s])
            wdB, bitB = rs_start(w_inB, rstB, SB, dB,
                                 rs_sendB.at[s], rs_recvB.at[s])
            rs_finish(wdA, bitA, w_inA, rstA, w_outA, SA)
            rs_finish(wdB, bitB, w_inB, rstB, w_outB, SB)

        def ag_start(cur, gbuf, sh, d, send_sem, recv_sem):
            bit0 = bit_is0(d)
            @pl.when(bit0)
            def _():
                gbuf[pl.ds(0, sh), :] = cur[...]
                pltpu.make_async_remote_copy(
                    src_ref=cur, dst_ref=gbuf.at[pl.ds(0, sh)],
                    send_sem=send_sem, recv_sem=recv_sem,
                    device_id=(my ^ d,), device_id_type=pl.DeviceIdType.MESH,
                ).start()
            @pl.when(jnp.logical_not(bit0))
            def _():
                gbuf[pl.ds(sh, sh), :] = cur[...]
                pltpu.make_async_remote_copy(
                    src_ref=cur, dst_ref=gbuf.at[pl.ds(sh, sh)],
                    send_sem=send_sem, recv_sem=recv_sem,
                    device_id=(my ^ d,), device_id_type=pl.DeviceIdType.MESH,
                ).start()
            return pltpu.make_async_remote_copy(
                src_ref=cur, dst_ref=gbuf.at[pl.ds(0, sh)],
                send_sem=send_sem, recv_sem=recv_sem,
                device_id=(my ^ d,), device_id_type=pl.DeviceIdType.MESH,
            )

        AG_A = [(w4A, g3A, R // 16, 8), (g3A, g2A, R // 8, 4),
                (g2A, g1A, R // 4, 2), (g1A, g0A, R // 2, 1)]
        AG_B = [(w4B, g3B, R // 16, 2), (g3B, g2B, R // 8, 1),
                (g2B, g1B, R // 4, 8), (g1B, g0B, R // 2, 4)]
        for j in range(4):
            curA, gbufA, shA, dA = AG_A[j]
            curB, gbufB, shB, dB = AG_B[j]
            wdA = ag_start(curA, gbufA, shA, dA, ag_sendA.at[j], ag_recvA.at[j])
            wdB = ag_start(curB, gbufB, shB, dB, ag_sendB.at[j], ag_recvB.at[j])
            wdA.wait_recv()
            wdB.wait_recv()
            drains.append(wdA)
            drains.append(wdB)

        out_cps = []
        for st, (g0, col0) in enumerate([(g0A, 0), (g0B, DC)]):
            for b in range(B):
                cp = pltpu.make_async_copy(
                    g0.at[pl.ds(b * SQ, SQ)],
                    out_any.at[b, :, pl.ds(col0, DC)],
                    out_sems.at[st * B + b])
                cp.start()
                out_cps.append(cp)
        for wd in drains:
            wd.wait_send()
        for cp in out_cps:
            cp.wait()

    def stream_bufs():
        return [
            pltpu.VMEM((R, DC), jnp.float32),
            pltpu.VMEM((R // 2, DC), jnp.float32),
            pltpu.VMEM((R // 4, DC), jnp.float32),
            pltpu.VMEM((R // 8, DC), jnp.float32),
            pltpu.VMEM((R // 16, DC), jnp.float32),
            pltpu.VMEM((R // 2, DC), jnp.float32),
            pltpu.VMEM((R // 4, DC), jnp.float32),
            pltpu.VMEM((R // 8, DC), jnp.float32),
            pltpu.VMEM((R // 16, DC), jnp.float32),
            pltpu.VMEM((R // 8, DC), jnp.float32),
            pltpu.VMEM((R // 4, DC), jnp.float32),
            pltpu.VMEM((R // 2, DC), jnp.float32),
            pltpu.VMEM((R, DC), jnp.float32),
        ]

    return pl.pallas_call(
        body,
        out_shape=jax.ShapeDtypeStruct((B, SQ, D), jnp.float32),
        in_specs=[pl.BlockSpec(memory_space=pl.ANY)] * 5,
        out_specs=pl.BlockSpec(memory_space=pl.ANY),
        scratch_shapes=(
            [
                pltpu.VMEM((B, SQ, D), jnp.float32),
                pltpu.VMEM((D, D), jnp.float32),
                pltpu.VMEM((D, D), jnp.float32),
                pltpu.VMEM((B, SQ, HKV * DH), jnp.float32),
                pltpu.VMEM((B, SQ, HKV * DH), jnp.float32),
                pltpu.VMEM((B, SQ, D), jnp.float32),
            ]
            + stream_bufs() + stream_bufs()
            + [
                pltpu.SemaphoreType.DMA((5,)),
                pltpu.SemaphoreType.DMA((4,)),
            ]
            + [pltpu.SemaphoreType.DMA((4,))] * 8
        ),
        compiler_params=pltpu.CompilerParams(collective_id=0),
    )(x, Wq, Wo, K_loc, V_loc)


# device time: 29859 ns/iter; 1.1941x vs baseline; 1.1941x over previous
import jax
import jax.numpy as jnp
from jax import lax
from jax.experimental import pallas as pl
from jax.experimental.pallas import tpu as pltpu

N_DEV = 16
B = 2
SQ = 128
D = 512
HQ_LOC = 8
DH = 64
GQA = 4
HKV = HQ_LOC // GQA
R = B * SQ
DC = D // 2


def kernel(x, Wq, Wo, K_ext, V_ext):
    idx = lax.axis_index("i")
    K_loc = jnp.reshape(
        lax.dynamic_slice_in_dim(K_ext, idx * HKV, HKV, axis=2), (B, SQ, HKV * DH))
    V_loc = jnp.reshape(
        lax.dynamic_slice_in_dim(V_ext, idx * HKV, HKV, axis=2), (B, SQ, HKV * DH))

    def body(x_ref, wq_ref, wo_ref, k_ref, v_ref, out_ref, att_ref,
             pstA, rst0A, wmidA, rst1A, wfinA, gmidA,
             pstB, rst0B, wmidB, rst1B, wfinB, gmidB,
             own_sems,
             rs_sendA, rs_recvA, ag_sendA, ag_recvA,
             rs_sendB, rs_recvB, ag_sendB, ag_recvB):
        my = lax.axis_index("i")

        barrier_sem = pltpu.get_barrier_semaphore()
        for d in (1, 2, 3, 4, 8, 12):
            pl.semaphore_signal(
                barrier_sem, inc=1,
                device_id=(my ^ d,), device_id_type=pl.DeviceIdType.MESH,
            )

        for b in range(B):
            qb = jnp.dot(x_ref[b], wq_ref[...],
                         preferred_element_type=jnp.float32)
            for h in range(HQ_LOC):
                c = h // GQA
                kb = k_ref[b, :, c * DH:(c + 1) * DH]
                vb = v_ref[b, :, c * DH:(c + 1) * DH]
                qh = qb[:, h * DH:(h + 1) * DH]
                s = lax.dot_general(
                    qh, kb, (((1,), (1,)), ((), ())),
                    preferred_element_type=jnp.float32,
                ) * 0.125
                m = jnp.max(s, axis=-1, keepdims=True)
                p = jnp.exp(s - m)
                l = jnp.sum(p, axis=-1, keepdims=True)
                o = jnp.dot(p, vb, preferred_element_type=jnp.float32) / l
                att_ref[b, :, h * DH:(h + 1) * DH] = o
            part = jnp.dot(att_ref[b], wo_ref[...],
                           preferred_element_type=jnp.float32)
            pstA[pl.ds(b * SQ, SQ), :] = part[:, :DC]
            pstB[pl.ds(b * SQ, SQ), :] = part[:, DC:]

        pl.semaphore_wait(barrier_sem, 6)

        drains = []

        def rs4_start(w_in, rst, S, dbase, send_sems, recv_sems, base):
            blk = S // 4
            t = lax.rem(my // dbase, 4)
            wds = []
            for j in (1, 2, 3):
                partner = my ^ (j * dbase)
                ts = t ^ j
                pltpu.make_async_remote_copy(
                    src_ref=w_in.at[pl.ds(ts * blk, blk)],
                    dst_ref=rst.at[j - 1],
                    send_sem=send_sems.at[base + j - 1],
                    recv_sem=recv_sems.at[base + j - 1],
                    device_id=(partner,), device_id_type=pl.DeviceIdType.MESH,
                ).start()
                wds.append(pltpu.make_async_remote_copy(
                    src_ref=rst.at[j - 1], dst_ref=rst.at[j - 1],
                    send_sem=send_sems.at[base + j - 1],
                    recv_sem=recv_sems.at[base + j - 1],
                    device_id=(partner,), device_id_type=pl.DeviceIdType.MESH,
                ))
            return wds, t, blk

        def rs4_finish(wds, t, blk, w_in, rst, w_out):
            for wd in wds:
                wd.wait_recv()
            w_out[...] = (w_in[pl.ds(t * blk, blk), :]
                          + rst[0] + rst[1] + rst[2])
            drains.extend(wds)

        def ag4_start(cur, slicer, blk, dbase, send_sems, recv_sems, base,
                      own_sem):
            t = lax.rem(my // dbase, 4)
            own = pltpu.make_async_copy(cur, slicer(t), own_sem)
            own.start()
            wds = []
            for j in (1, 2, 3):
                partner = my ^ (j * dbase)
                pltpu.make_async_remote_copy(
                    src_ref=cur, dst_ref=slicer(t),
                    send_sem=send_sems.at[base + j - 1],
                    recv_sem=recv_sems.at[base + j - 1],
                    device_id=(partner,), device_id_type=pl.DeviceIdType.MESH,
                ).start()
                wds.append(pltpu.make_async_remote_copy(
                    src_ref=cur, dst_ref=slicer(0),
                    send_sem=send_sems.at[base + j - 1],
                    recv_sem=recv_sems.at[base + j - 1],
                    device_id=(partner,), device_id_type=pl.DeviceIdType.MESH,
                ))
            return wds, own

        def ag4_finish(wds, own):
            for wd in wds:
                wd.wait_recv()
            own.wait()
            drains.extend(wds)

        wdsA, tA, blkA = rs4_start(pstA, rst0A, R, 1, rs_sendA, rs_recvA, 0)
        wdsB, tB, blkB = rs4_start(pstB, rst0B, R, 4, rs_sendB, rs_recvB, 0)
        rs4_finish(wdsA, tA, blkA, pstA, rst0A, wmidA)
        rs4_finish(wdsB, tB, blkB, pstB, rst0B, wmidB)
        wdsA, tA, blkA = rs4_start(wmidA, rst1A, R // 4, 4, rs_sendA, rs_recvA, 3)
        wdsB, tB, blkB = rs4_start(wmidB, rst1B, R // 4, 1, rs_sendB, rs_recvB, 3)
        rs4_finish(wdsA, tA, blkA, wmidA, rst1A, wfinA)
        rs4_finish(wdsB, tB, blkB, wmidB, rst1B, wfinB)

        blk1 = R // 16
        wdsA, ownA = ag4_start(
            wfinA, lambda t: gmidA.at[pl.ds(t * blk1, blk1)], blk1, 4,
            ag_sendA, ag_recvA, 0, own_sems.at[0])
        wdsB, ownB = ag4_start(
            wfinB, lambda t: gmidB.at[pl.ds(t * blk1, blk1)], blk1, 1,
            ag_sendB, ag_recvB, 0, own_sems.at[1])
        ag4_finish(wdsA, ownA)
        ag4_finish(wdsB, ownB)

        blk0 = R // 4
        def out_slicer(col0):
            return lambda t: out_ref.at[
                t // 2, pl.ds(lax.rem(t, 2) * blk0, blk0), pl.ds(col0, DC)]
        wdsA, ownA = ag4_start(
            gmidA, out_slicer(0), blk0, 1,
            ag_sendA, ag_recvA, 3, own_sems.at[2])
        wdsB, ownB = ag4_start(
            gmidB, out_slicer(DC), blk0, 4,
            ag_sendB, ag_recvB, 3, own_sems.at[3])
        ag4_finish(wdsA, ownA)
        ag4_finish(wdsB, ownB)

        for wd in drains:
            wd.wait_send()

    def stream_bufs():
        return [
            pltpu.VMEM((R, DC), jnp.float32),
            pltpu.VMEM((3, R // 4, DC), jnp.float32),
            pltpu.VMEM((R // 4, DC), jnp.float32),
            pltpu.VMEM((3, R // 16, DC), jnp.float32),
            pltpu.VMEM((R // 16, DC), jnp.float32),
            pltpu.VMEM((R // 4, DC), jnp.float32),
        ]

    return pl.pallas_call(
        body,
        out_shape=jax.ShapeDtypeStruct((B, SQ, D), jnp.float32),
        in_specs=[pl.BlockSpec(memory_space=pltpu.VMEM)] * 5,
        out_specs=pl.BlockSpec(memory_space=pltpu.VMEM),
        scratch_shapes=(
            [pltpu.VMEM((B, SQ, D), jnp.float32)]
            + stream_bufs() + stream_bufs()
            + [pltpu.SemaphoreType.DMA((4,))]
            + [pltpu.SemaphoreType.DMA((6,))] * 8
        ),
        compiler_params=pltpu.CompilerParams(collective_id=0),
    )(x, Wq, Wo, K_loc, V_loc)
